# Initial kernel scaffold; baseline (speedup 1.0000x reference)
import jax
import jax.numpy as jnp
from jax.experimental import pallas as pl


def kernel(encoded_tokens, pos_table):
    raise NotImplementedError("write your pallas kernel here")


# speedup vs baseline: 1.4930x; 1.4930x over previous
import jax
import jax.numpy as jnp
from jax.experimental import pallas as pl


def _add_kernel(x_ref, p_ref, o_ref):
    o_ref[...] = x_ref[...] + p_ref[...]


def kernel(encoded_tokens, pos_table):
    B, N, E = encoded_tokens.shape
    TBLK = 512
    grid = (N // TBLK, B)
    return pl.pallas_call(
        _add_kernel,
        grid=grid,
        in_specs=[
            pl.BlockSpec((None, TBLK, E), lambda t, b: (b, t, 0)),
            pl.BlockSpec((TBLK, E), lambda t, b: (t, 0)),
        ],
        out_specs=pl.BlockSpec((None, TBLK, E), lambda t, b: (b, t, 0)),
        out_shape=jax.ShapeDtypeStruct((B, N, E), encoded_tokens.dtype),
    )(encoded_tokens, pos_table)


# speedup vs baseline: 1.7378x; 1.1640x over previous
import jax
import jax.numpy as jnp
from jax.experimental import pallas as pl


def _add_kernel(x_ref, p_ref, o_ref):
    o_ref[...] = x_ref[...] + p_ref[...]


def kernel(encoded_tokens, pos_table):
    B, N, E = encoded_tokens.shape
    TBLK = 2048
    grid = (N // TBLK, B)
    return pl.pallas_call(
        _add_kernel,
        grid=grid,
        in_specs=[
            pl.BlockSpec((None, TBLK, E), lambda t, b: (b, t, 0)),
            pl.BlockSpec((TBLK, E), lambda t, b: (t, 0)),
        ],
        out_specs=pl.BlockSpec((None, TBLK, E), lambda t, b: (b, t, 0)),
        out_shape=jax.ShapeDtypeStruct((B, N, E), encoded_tokens.dtype),
    )(encoded_tokens, pos_table)
